# baseline (speedup 1.0000x reference)
import jax
import jax.numpy as jnp
from jax import lax
from jax.experimental import pallas as pl
from jax.experimental.pallas import tpu as pltpu
from jax.experimental.pallas import tpu_sc as plsc

_X, _Y, _Z = 128, 128, 32
_R = _X * _Z
_PL = _Z * _Y
_K = 10
_NEG = float("-inf")
_BIG = 2**30


def _nms_kernel(x_ref, nms_ref, rv_ref):
    a = x_ref[0]
    rmod = jnp.bitwise_and(lax.broadcasted_iota(jnp.int32, (_R, 128), 0), _Z - 1)
    neg_row = jnp.full((1, 128), _NEG, jnp.float32)
    neg_col = jnp.full((_R, 1), _NEG, jnp.float32)
    neg_32r = jnp.full((32, 128), _NEG, jnp.float32)

    zp = jnp.concatenate([a[1:], neg_row], axis=0)
    zp = jnp.where(rmod == _Z - 1, _NEG, zp)
    zm = jnp.concatenate([neg_row, a[:-1]], axis=0)
    zm = jnp.where(rmod == 0, _NEG, zm)
    mz = jnp.maximum(a, jnp.maximum(zp, zm))
    yp = jnp.concatenate([mz[:, 1:], neg_col], axis=1)
    ym = jnp.concatenate([neg_col, mz[:, :-1]], axis=1)
    my = jnp.maximum(mz, jnp.maximum(yp, ym))
    xp = jnp.concatenate([my[32:], neg_32r], axis=0)
    xm = jnp.concatenate([neg_32r, my[:-32]], axis=0)
    m = jnp.maximum(my, jnp.maximum(xp, xm))

    nms = jnp.where(a == m, a, 0.0)
    nms_ref[0] = nms
    m1 = jnp.max(nms.reshape(_X, _Z, 128), axis=1)
    rv_ref[0] = jnp.max(m1, axis=1, keepdims=True)


def _sc_topk(nms_hbm, rv_hbm, out_hbm, rv_v, plane_v, out_v):
    wid = lax.axis_index("s") * 2 + lax.axis_index("c")
    lane = lax.iota(jnp.int32, 16)
    pltpu.sync_copy(rv_hbm.at[wid], rv_v)

    neg = jnp.full((16,), _NEG, jnp.float32)
    big = jnp.full((16,), _BIG, jnp.int32)
    zero_i = jnp.zeros((16,), jnp.int32)
    _gdn = lax.GatherDimensionNumbers(
        offset_dims=(), collapsed_slice_dims=(0,), start_index_map=(0,)
    )

    def shuf(v, idx):
        return lax.gather(
            v, idx[:, None], _gdn, (1,),
            mode=lax.GatherScatterMode.PROMISE_IN_BOUNDS,
        )

    def bfly(v, op):
        for s in (1, 2, 4, 8):
            v = op(v, shuf(v, lane ^ s))
        return v

    vals, xs, keys = [], [], []
    for _ in range(_K):
        bestv, besti = neg, big
        for k in range(_X // 16):
            v = rv_v[pl.ds(k * 16, 16)]
            take = v > bestv
            bestv = jnp.where(take, v, bestv)
            besti = jnp.where(take, lane + k * 16, besti)
        m = bfly(bestv, jnp.maximum)
        xw = bfly(jnp.where(bestv == m, besti, _BIG), jnp.minimum)
        x_s = xw[0]

        pltpu.sync_copy(nms_hbm.at[pl.ds((wid * _X + x_s) * _PL, _PL)], plane_v)
        for xj, kj in zip(xs, keys):
            same = x_s == xj[0]
            off = jnp.bitwise_and(kj[0], _Z - 1) * 128 + lax.shift_right_logical(
                kj[0], 5
            )
            offok = jnp.where(same, off, 0)
            match = jnp.where(same, off, -1)
            cb = (offok // 16) * 16
            chunk = plane_v[pl.ds(cb, 16)]
            plane_v[pl.ds(cb, 16)] = jnp.where(lane + cb == match, _NEG, chunk)

        def scan_body(k8, carry):
            k1, cnt, vless = carry
            for j in range(8):
                idx = k8 * 8 + j
                v = plane_v[pl.ds(idx * 16, 16)]
                key = (jnp.bitwise_and(idx, 7) * 16 + lane) * _Z + (
                    lax.shift_right_logical(idx, 3)
                )
                ism = v == m
                k1 = jnp.minimum(k1, jnp.where(ism, key, _BIG))
                cnt = cnt + jnp.where(ism, 1, 0)
                vless = jnp.maximum(vless, jnp.where(ism, _NEG, v))
            return k1, cnt, vless

        k1, cnt, vless = lax.fori_loop(
            0, _PL // 128, scan_body, (big, zero_i, neg)
        )
        kq = bfly(k1, jnp.minimum)
        ctot = bfly(cnt, jnp.add)
        vl = bfly(vless, jnp.maximum)
        newrv = jnp.where(ctot >= 2, m, vl)

        vals.append(m)
        xs.append(xw)
        keys.append(kq)

        rb = (x_s // 16) * 16
        sel = lane + rb == xw
        rv_v[pl.ds(rb, 16)] = jnp.where(sel, newrv, rv_v[pl.ds(rb, 16)])

    def lanevec(splats, dtype):
        out = jnp.zeros((16,), dtype)
        for i, s in enumerate(splats):
            out = jnp.where(lane == i, s.astype(dtype), out)
        return out

    fv = lanevec(vals, jnp.float32)
    xv = lanevec(xs, jnp.int32)
    kv = lanevec(keys, jnp.int32)
    iy = lax.shift_right_logical(kv, 5)
    iz = jnp.bitwise_and(kv, _Z - 1)
    keep = lane < _K
    locx = (xv.astype(jnp.float32) / float(_X - 1) * 8000.0 + 0.0) - 4000.0
    locy = (iy.astype(jnp.float32) / float(_Y - 1) * 8000.0 + 0.0) - 4000.0
    locz = (iz.astype(jnp.float32) / float(_Z - 1) * 2000.0 + 800.0) - 1000.0
    flag = jnp.where(fv > 0.3, 0.0, -1.0)
    for f, vec in enumerate([locx, locy, locz, flag, fv]):
        out_v[pl.ds(f * 16, 16)] = jnp.where(keep, vec, 0.0)
    pltpu.sync_copy(out_v, out_hbm.at[wid])


@jax.jit
def kernel(root_cubes):
    rc = lax.stop_gradient(root_cubes)
    b = rc.shape[0]
    a2 = rc.transpose(0, 1, 3, 2).reshape(b, _R, 128)
    nms, rv = pl.pallas_call(
        _nms_kernel,
        grid=(b,),
        in_specs=[pl.BlockSpec((1, _R, 128), lambda i: (i, 0, 0))],
        out_specs=[
            pl.BlockSpec((1, _R, 128), lambda i: (i, 0, 0)),
            pl.BlockSpec((1, _X, 1), lambda i: (i, 0, 0)),
        ],
        out_shape=[
            jax.ShapeDtypeStruct((b, _R, 128), jnp.float32),
            jax.ShapeDtypeStruct((b, _X, 1), jnp.float32),
        ],
    )(a2)

    mesh = plsc.VectorSubcoreMesh(core_axis_name="c", subcore_axis_name="s")
    out = pl.kernel(
        _sc_topk,
        mesh=mesh,
        out_type=jax.ShapeDtypeStruct((b, 80), jnp.float32),
        scratch_types=[
            pltpu.VMEM((_X,), jnp.float32),
            pltpu.VMEM((_PL,), jnp.float32),
            pltpu.VMEM((80,), jnp.float32),
        ],
    )(nms.reshape(b * _R * 128), rv.reshape(b, _X))
    return out.reshape(b, 5, 16)[:, :, :_K].transpose(0, 2, 1)
